# baseline (speedup 1.0000x reference)
import jax
import jax.numpy as jnp
from jax import lax
from jax.experimental import pallas as pl
from jax.experimental.pallas import tpu as pltpu
from jax.experimental.pallas import tpu_sc as plsc

NUM_FIELDS = 26
CARD = 100000
EMBED = 50

B, T = 1024, 20
NBT = B * T
PAIRS = NUM_FIELDS * EMBED

NC, NS = 2, 16
NW = NC * NS
BASE_PAIRS = PAIRS // NW
EXTRA = PAIRS - BASE_PAIRS * NW

OCHUNK = 4096
NCHUNKS = NBT // OCHUNK


def _emb_body(wt_hbm, xt_hbm, out_hbm, row_v, idx_v, out_v, sem):
    wid = lax.axis_index("s") * NC + lax.axis_index("c")
    p0 = wid * BASE_PAIRS + jnp.minimum(wid, EXTRA)
    cnt = BASE_PAIRS + jnp.where(wid < EXTRA, 1, 0)

    def pair_body(k, prev_i):
        p = p0 + k
        i = p // EMBED
        e = p % EMBED

        @pl.when(i != prev_i)
        def _():
            pltpu.sync_copy(xt_hbm.at[i, pl.ds(0, NBT)], idx_v)

        pltpu.sync_copy(wt_hbm.at[i, e, pl.ds(0, CARD)], row_v)

        def chunk_body(c, carry):
            cbase = c * OCHUNK

            @plsc.parallel_loop(0, OCHUNK // 16, 1, unroll=8)
            def _gat(k16):
                vidx = idx_v[pl.ds(cbase + k16 * 16, 16)]
                out_v[pl.ds(k16 * 16, 16)] = plsc.load_gather(row_v, [vidx])
            def trow(j, carry3):
                pltpu.sync_copy(
                    out_v.at[pl.ds(j * B, B)],
                    out_hbm.at[(c * 4 + j) * PAIRS + p, pl.ds(0, B)],
                )
                return carry3

            lax.fori_loop(0, 4, trow, 0)
            return carry

        lax.fori_loop(0, NCHUNKS, chunk_body, 0)
        return i

    lax.fori_loop(0, cnt, pair_body, jnp.int32(-1))


@jax.jit
def _emb_gather(wt, xt):
    mesh = plsc.VectorSubcoreMesh(core_axis_name="c", subcore_axis_name="s")
    return pl.kernel(
        _emb_body,
        out_type=jax.ShapeDtypeStruct((T * PAIRS, B), jnp.float32),
        mesh=mesh,
        scratch_types=[
            pltpu.VMEM((CARD,), jnp.float32),
            pltpu.VMEM((NBT,), jnp.int32),
            pltpu.VMEM((OCHUNK,), jnp.float32),
            pltpu.SemaphoreType.DMA,
        ],
        compiler_params=pltpu.CompilerParams(needs_layout_passes=False),
    )(wt, xt)


def kernel(x, W):
    wt = W.transpose(0, 2, 1)
    xt = x.reshape(NBT, NUM_FIELDS).astype(jnp.int32)
    xt = xt.T.reshape(NUM_FIELDS, B, T).transpose(0, 2, 1).reshape(
        NUM_FIELDS, NBT
    )
    out = _emb_gather(wt, xt)
    return lax.reshape(out, (B, T, PAIRS, 1), dimensions=(1, 0))


# speedup vs baseline: 1.2848x; 1.2848x over previous
import jax
import jax.numpy as jnp
from jax import lax
from jax.experimental import pallas as pl
from jax.experimental.pallas import tpu as pltpu
from jax.experimental.pallas import tpu_sc as plsc

NUM_FIELDS = 26
CARD = 100000
EMBED = 50

B, T = 1024, 20
NBT = B * T
PAIRS = NUM_FIELDS * EMBED

NC, NS = 2, 16
NW = NC * NS
BASE_PAIRS = PAIRS // NW
EXTRA = PAIRS - BASE_PAIRS * NW

OCHUNK = 4096
NCHUNKS = NBT // OCHUNK


def _emb_body(wt_hbm, xt_hbm, out_hbm, row_v, idx_v, out_v0, out_v1, s0, s1):
    wid = lax.axis_index("s") * NC + lax.axis_index("c")
    p0 = wid * BASE_PAIRS + jnp.minimum(wid, EXTRA)
    cnt = BASE_PAIRS + jnp.where(wid < EXTRA, 1, 0)
    bufs = (out_v0, out_v1)
    sems = (s0, s1)

    def pair_body(k, prev_i):
        p = p0 + k
        i = p // EMBED
        e = p % EMBED

        @pl.when(i != prev_i)
        def _():
            pltpu.sync_copy(xt_hbm.at[i, pl.ds(0, NBT)], idx_v)

        pltpu.sync_copy(wt_hbm.at[i, e, pl.ds(0, CARD)], row_v)

        for c in range(NCHUNKS):
            out_v = bufs[c % 2]
            sem = sems[c % 2]
            dst = out_hbm.at[p, pl.ds(c * OCHUNK, OCHUNK)]
            if c >= 2:
                pltpu.make_async_copy(
                    out_hbm.at[p, pl.ds((c - 2) * OCHUNK, OCHUNK)], out_v, sem
                ).wait()
            cbase = c * OCHUNK

            @plsc.parallel_loop(0, OCHUNK // 16, 1, unroll=8)
            def _gat(k16):
                vidx = idx_v[pl.ds(cbase + k16 * 16, 16)]
                out_v[pl.ds(k16 * 16, 16)] = plsc.load_gather(row_v, [vidx])

            pltpu.async_copy(out_v, dst, sem)

        for c in (NCHUNKS - 2, NCHUNKS - 1):
            pltpu.make_async_copy(
                out_hbm.at[p, pl.ds(c * OCHUNK, OCHUNK)], bufs[c % 2], sems[c % 2]
            ).wait()
        return i

    lax.fori_loop(0, cnt, pair_body, jnp.int32(-1))


@jax.jit
def _emb_gather(wt, xt):
    mesh = plsc.VectorSubcoreMesh(core_axis_name="c", subcore_axis_name="s")
    return pl.kernel(
        _emb_body,
        out_type=jax.ShapeDtypeStruct((PAIRS, NBT), jnp.float32),
        mesh=mesh,
        scratch_types=[
            pltpu.VMEM((CARD,), jnp.float32),
            pltpu.VMEM((NBT,), jnp.int32),
            pltpu.VMEM((OCHUNK,), jnp.float32),
            pltpu.VMEM((OCHUNK,), jnp.float32),
            pltpu.SemaphoreType.DMA,
            pltpu.SemaphoreType.DMA,
        ],
        compiler_params=pltpu.CompilerParams(needs_layout_passes=False),
    )(wt, xt)


def kernel(x, W):
    wt = W.transpose(0, 2, 1)
    xt = x.reshape(NBT, NUM_FIELDS).astype(jnp.int32).T
    out = _emb_gather(wt, xt)
    return lax.reshape(out, (B, T, PAIRS, 1), dimensions=(1, 0))
